# baseline (speedup 1.0000x reference)
import dataclasses

import jax
import jax.numpy as jnp
import numpy as np
from jax import lax
from jax.experimental import pallas as pl
from jax.experimental.pallas import tpu as pltpu
from jax.experimental.pallas import tpu_sc as plsc

N = 10000
E = 320000
D = 128

NCORES = 2
NSUB = 16
NTILES = NCORES * NSUB
CW = 128
NCH = 80
E_PAD = NTILES * NCH * CW
NP = 10240
RPT = NP // NSUB

_mesh = plsc.VectorSubcoreMesh(core_axis_name="c", subcore_axis_name="s")


def _deg_body(dst_hbm, out_hbm, dstv, hist, sem):
    del sem
    cid = lax.axis_index("c")
    sid = lax.axis_index("s")
    tid = cid * NSUB + sid
    pltpu.sync_copy(dst_hbm.at[pl.ds(tid * NCH, NCH)], dstv)

    @pl.loop(0, NP // 128)
    def _(r):
        @pl.loop(0, 128 // 16)
        def _(j):
            hist[r, pl.ds(j * 16, 16)] = jnp.zeros((16,), jnp.float32)

    ones = jnp.ones((16,), jnp.float32)

    @pl.loop(0, NCH)
    def _(c):
        @pl.loop(0, CW // 16)
        def _(k):
            idx = dstv[c, pl.ds(k * 16, 16)]
            plsc.addupdate_scatter(hist, [lax.shift_right_logical(idx, 7),
                                          lax.bitwise_and(idx, 127)], ones)

    pltpu.sync_copy(hist, out_hbm.at[tid])


_deg_cp = pltpu.CompilerParams()
if "needs_layout_passes" in pltpu.CompilerParams.__dataclass_fields__:
    _deg_cp = dataclasses.replace(_deg_cp, needs_layout_passes=False)

_deg_call = pl.kernel(
    _deg_body,
    out_type=jax.ShapeDtypeStruct((NTILES, NP // 128, 128), jnp.float32),
    mesh=_mesh,
    compiler_params=_deg_cp,
    scratch_types=[
        pltpu.VMEM((NCH, CW), jnp.int32),
        pltpu.VMEM((NP // 128, 128), jnp.float32),
        pltpu.SemaphoreType.DMA,
    ],
)


HCH = NCH // 2
SUB = 1
SW = CW // SUB


def _start_gather(y_hbm, srcv, c, buf, sem):
    for k in range(SUB):
        pltpu.async_copy(y_hbm.at[srcv.at[c, pl.ds(k * SW, SW)]],
                         buf.at[pl.ds(k * SW, SW)], sem)


def _wait_gather(y_hbm, srcv, c, buf, sem):
    for k in range(SUB):
        pltpu.make_async_copy(y_hbm.at[srcv.at[c, pl.ds(k * SW, SW)]],
                              buf.at[pl.ds(k * SW, SW)], sem).wait()


def _prop_body(y_hbm, src_hbm, dst_hbm, zrows_hbm, out_hbm,
               srcv, dstv, rows0, rows1, acc, g0, g1):
    cid = lax.axis_index("c")
    sid = lax.axis_index("s")
    tid = cid * NSUB + sid
    pltpu.sync_copy(zrows_hbm, acc.at[pl.ds(sid * RPT, RPT)])
    plsc.subcore_barrier()

    for ph in range(2):
        base = tid * NCH + ph * HCH
        pltpu.sync_copy(src_hbm.at[pl.ds(base, HCH)], srcv)
        pltpu.sync_copy(dst_hbm.at[pl.ds(base, HCH)], dstv)

        _start_gather(y_hbm, srcv, 0, rows0, g0)

        @pl.loop(0, HCH // 2 - 1)
        def _(i):
            c = i * 2
            _start_gather(y_hbm, srcv, c + 1, rows1, g1)
            _wait_gather(y_hbm, srcv, c, rows0, g0)
            pltpu.sync_copy(rows0, acc.at[dstv.at[c]], add=True)
            _start_gather(y_hbm, srcv, c + 2, rows0, g0)
            _wait_gather(y_hbm, srcv, c + 1, rows1, g1)
            pltpu.sync_copy(rows1, acc.at[dstv.at[c + 1]], add=True)

        _start_gather(y_hbm, srcv, HCH - 1, rows1, g1)
        _wait_gather(y_hbm, srcv, HCH - 2, rows0, g0)
        pltpu.sync_copy(rows0, acc.at[dstv.at[HCH - 2]], add=True)
        _wait_gather(y_hbm, srcv, HCH - 1, rows1, g1)
        pltpu.sync_copy(rows1, acc.at[dstv.at[HCH - 1]], add=True)

    plsc.subcore_barrier()
    pltpu.sync_copy(acc.at[pl.ds(sid * RPT, RPT)],
                    out_hbm.at[cid, pl.ds(sid * RPT, RPT)])


_prop_call = pl.kernel(
    _prop_body,
    out_type=jax.ShapeDtypeStruct((NCORES, NP, D), jnp.float32),
    mesh=_mesh,
    scratch_types=[
        pltpu.VMEM((HCH, CW), jnp.int32),
        pltpu.VMEM((HCH, CW), jnp.int32),
        pltpu.VMEM((CW, D), jnp.float32),
        pltpu.VMEM((CW, D), jnp.float32),
        pltpu.VMEM_SHARED((NP, D), jnp.float32),
        pltpu.SemaphoreType.DMA,
        pltpu.SemaphoreType.DMA,
    ],
)


ROWS_BLK = 5120


def _mm_body(x_ref, w_ref, y_ref):
    y_ref[...] = jnp.dot(x_ref[...], w_ref[...],
                         preferred_element_type=jnp.float32)


def _scale_body(deg_ref, xw_ref, y_ref):
    dinv = lax.rsqrt(deg_ref[...].reshape(ROWS_BLK, 1))
    y_ref[...] = xw_ref[...] * dinv


def _mid_body(deg_ref, p_ref, y1_ref, b1_ref, w2_ref, y2_ref):
    dinv = lax.rsqrt(deg_ref[...].reshape(ROWS_BLK, 1))
    z = p_ref[0] + p_ref[1] + y1_ref[...]
    h = jnp.maximum(z * dinv + b1_ref[...], 0.0)
    y2_ref[...] = jnp.dot(h, w2_ref[...],
                          preferred_element_type=jnp.float32) * dinv


def _out_body(deg_ref, p_ref, y2_ref, b2_ref, o_ref):
    dinv = lax.rsqrt(deg_ref[...].reshape(ROWS_BLK, 1))
    o_ref[...] = (p_ref[0] + p_ref[1] + y2_ref[...]) * dinv + b2_ref[...]


def _degp_spec():
    return pl.BlockSpec((ROWS_BLK,), lambda i: (i,))


def _rows_spec():
    return pl.BlockSpec((ROWS_BLK, D), lambda i: (i, 0))


def _p_spec():
    return pl.BlockSpec((NCORES, ROWS_BLK, D), lambda i: (0, i, 0))


def _full_spec():
    return pl.BlockSpec((D, D), lambda i: (0, 0))


def _bias_spec():
    return pl.BlockSpec((1, D), lambda i: (0, 0))


_mm_call = pl.pallas_call(
    _mm_body,
    grid=(NP // ROWS_BLK,),
    in_specs=[_rows_spec(), _full_spec()],
    out_specs=_rows_spec(),
    out_shape=jax.ShapeDtypeStruct((NP, D), jnp.float32),
)

_scale_call = pl.pallas_call(
    _scale_body,
    grid=(NP // ROWS_BLK,),
    in_specs=[_degp_spec(), _rows_spec()],
    out_specs=_rows_spec(),
    out_shape=jax.ShapeDtypeStruct((NP, D), jnp.float32),
)

_mid_call = pl.pallas_call(
    _mid_body,
    grid=(NP // ROWS_BLK,),
    in_specs=[_degp_spec(), _p_spec(), _rows_spec(), _bias_spec(), _full_spec()],
    out_specs=_rows_spec(),
    out_shape=jax.ShapeDtypeStruct((NP, D), jnp.float32),
)

_out_call = pl.pallas_call(
    _out_body,
    grid=(NP // ROWS_BLK,),
    in_specs=[_degp_spec(), _p_spec(), _rows_spec(), _bias_spec()],
    out_specs=_rows_spec(),
    out_shape=jax.ShapeDtypeStruct((N, D), jnp.float32),
)


def kernel(x, edge_index, W1, b1, W2, b2):
    pad2 = jnp.asarray(N + (np.arange(E_PAD - E, dtype=np.int32) % (NP - N))
                       ).reshape((E_PAD - E) // CW, CW)
    src2 = jnp.concatenate([edge_index[0].reshape(E // CW, CW), pad2])
    dst2 = jnp.concatenate([edge_index[1].reshape(E // CW, CW), pad2])
    zrows = jnp.zeros((RPT, D), jnp.float32)

    xw1 = _mm_call(x, W1)
    degp = _deg_call(dst2)
    deg = degp.sum(axis=0).reshape(NP) + 1.0
    y1 = _scale_call(deg, xw1)
    p1 = _prop_call(y1, src2, dst2, zrows)
    y2 = _mid_call(deg, p1, y1, b1.reshape(1, D), W2)
    p2 = _prop_call(y2, src2, dst2, zrows)
    return _out_call(deg, p2, y2, b2.reshape(1, D))


# speedup vs baseline: 1.0044x; 1.0044x over previous
import dataclasses

import jax
import jax.numpy as jnp
import numpy as np
from jax import lax
from jax.experimental import pallas as pl
from jax.experimental.pallas import tpu as pltpu
from jax.experimental.pallas import tpu_sc as plsc

N = 10000
E = 320000
D = 128

NCORES = 2
NSUB = 16
NTILES = NCORES * NSUB
CW = 128
NCH = 80
E_PAD = NTILES * NCH * CW
NP = 10240
RPT = NP // NSUB

_mesh = plsc.VectorSubcoreMesh(core_axis_name="c", subcore_axis_name="s")


EPT = E // NTILES


def _deg_body(dst_hbm, out_hbm, dstv, hist, sem):
    del sem
    cid = lax.axis_index("c")
    sid = lax.axis_index("s")
    tid = cid * NSUB + sid
    pltpu.sync_copy(dst_hbm.at[pl.ds(tid * EPT, EPT)], dstv)

    @pl.loop(0, NP // 128)
    def _(r):
        @pl.loop(0, 128 // 16)
        def _(j):
            hist[r, pl.ds(j * 16, 16)] = jnp.zeros((16,), jnp.float32)

    ones = jnp.ones((16,), jnp.float32)

    @pl.loop(0, EPT // 16)
    def _(v):
        idx = dstv[pl.ds(v * 16, 16)]
        plsc.addupdate_scatter(hist, [lax.shift_right_logical(idx, 7),
                                      lax.bitwise_and(idx, 127)], ones)

    pltpu.sync_copy(hist, out_hbm.at[tid])


_deg_cp = pltpu.CompilerParams()
if "needs_layout_passes" in pltpu.CompilerParams.__dataclass_fields__:
    _deg_cp = dataclasses.replace(_deg_cp, needs_layout_passes=False)

_deg_call = pl.kernel(
    _deg_body,
    out_type=jax.ShapeDtypeStruct((NTILES, NP // 128, 128), jnp.float32),
    mesh=_mesh,
    compiler_params=_deg_cp,
    scratch_types=[
        pltpu.VMEM((EPT,), jnp.int32),
        pltpu.VMEM((NP // 128, 128), jnp.float32),
        pltpu.SemaphoreType.DMA,
    ],
)


HCH = NCH // 2
SUB = 1
SW = CW // SUB


def _start_gather(y_hbm, srcv, c, buf, sem):
    for k in range(SUB):
        pltpu.async_copy(y_hbm.at[srcv.at[c, pl.ds(k * SW, SW)]],
                         buf.at[pl.ds(k * SW, SW)], sem)


def _wait_gather(y_hbm, srcv, c, buf, sem):
    for k in range(SUB):
        pltpu.make_async_copy(y_hbm.at[srcv.at[c, pl.ds(k * SW, SW)]],
                              buf.at[pl.ds(k * SW, SW)], sem).wait()


def _prop_body(y_hbm, src_hbm, dst_hbm, zrows_hbm, out_hbm,
               srcv, dstv, rows0, rows1, acc, g0, g1):
    cid = lax.axis_index("c")
    sid = lax.axis_index("s")
    tid = cid * NSUB + sid
    pltpu.sync_copy(zrows_hbm, acc.at[pl.ds(sid * RPT, RPT)])
    plsc.subcore_barrier()

    for ph in range(2):
        base = tid * NCH + ph * HCH
        pltpu.sync_copy(src_hbm.at[pl.ds(base, HCH)], srcv)
        pltpu.sync_copy(dst_hbm.at[pl.ds(base, HCH)], dstv)

        _start_gather(y_hbm, srcv, 0, rows0, g0)

        @pl.loop(0, HCH // 2 - 1)
        def _(i):
            c = i * 2
            _start_gather(y_hbm, srcv, c + 1, rows1, g1)
            _wait_gather(y_hbm, srcv, c, rows0, g0)
            pltpu.sync_copy(rows0, acc.at[dstv.at[c]], add=True)
            _start_gather(y_hbm, srcv, c + 2, rows0, g0)
            _wait_gather(y_hbm, srcv, c + 1, rows1, g1)
            pltpu.sync_copy(rows1, acc.at[dstv.at[c + 1]], add=True)

        _start_gather(y_hbm, srcv, HCH - 1, rows1, g1)
        _wait_gather(y_hbm, srcv, HCH - 2, rows0, g0)
        pltpu.sync_copy(rows0, acc.at[dstv.at[HCH - 2]], add=True)
        _wait_gather(y_hbm, srcv, HCH - 1, rows1, g1)
        pltpu.sync_copy(rows1, acc.at[dstv.at[HCH - 1]], add=True)

    plsc.subcore_barrier()
    pltpu.sync_copy(acc.at[pl.ds(sid * RPT, RPT)],
                    out_hbm.at[cid, pl.ds(sid * RPT, RPT)])


_prop_call = pl.kernel(
    _prop_body,
    out_type=jax.ShapeDtypeStruct((NCORES, NP, D), jnp.float32),
    mesh=_mesh,
    scratch_types=[
        pltpu.VMEM((HCH, CW), jnp.int32),
        pltpu.VMEM((HCH, CW), jnp.int32),
        pltpu.VMEM((CW, D), jnp.float32),
        pltpu.VMEM((CW, D), jnp.float32),
        pltpu.VMEM_SHARED((NP, D), jnp.float32),
        pltpu.SemaphoreType.DMA,
        pltpu.SemaphoreType.DMA,
    ],
)


ROWS_BLK = 5120


def _lin1_body(deg_ref, x_ref, w_ref, y_ref):
    dinv = lax.rsqrt(deg_ref[...].reshape(ROWS_BLK, 1))
    y_ref[...] = jnp.dot(x_ref[...], w_ref[...],
                         preferred_element_type=jnp.float32) * dinv


def _mid_body(deg_ref, p_ref, y1_ref, b1_ref, w2_ref, y2_ref):
    dinv = lax.rsqrt(deg_ref[...].reshape(ROWS_BLK, 1))
    z = p_ref[0] + p_ref[1] + y1_ref[...]
    h = jnp.maximum(z * dinv + b1_ref[...], 0.0)
    y2_ref[...] = jnp.dot(h, w2_ref[...],
                          preferred_element_type=jnp.float32) * dinv


def _out_body(deg_ref, p_ref, y2_ref, b2_ref, o_ref):
    dinv = lax.rsqrt(deg_ref[...].reshape(ROWS_BLK, 1))
    o_ref[...] = (p_ref[0] + p_ref[1] + y2_ref[...]) * dinv + b2_ref[...]


def _degp_spec():
    return pl.BlockSpec((ROWS_BLK,), lambda i: (i,))


def _rows_spec():
    return pl.BlockSpec((ROWS_BLK, D), lambda i: (i, 0))


def _p_spec():
    return pl.BlockSpec((NCORES, ROWS_BLK, D), lambda i: (0, i, 0))


def _full_spec():
    return pl.BlockSpec((D, D), lambda i: (0, 0))


def _bias_spec():
    return pl.BlockSpec((1, D), lambda i: (0, 0))


_lin1_call = pl.pallas_call(
    _lin1_body,
    grid=(NP // ROWS_BLK,),
    in_specs=[_degp_spec(), _rows_spec(), _full_spec()],
    out_specs=_rows_spec(),
    out_shape=jax.ShapeDtypeStruct((NP, D), jnp.float32),
)

_mid_call = pl.pallas_call(
    _mid_body,
    grid=(NP // ROWS_BLK,),
    in_specs=[_degp_spec(), _p_spec(), _rows_spec(), _bias_spec(), _full_spec()],
    out_specs=_rows_spec(),
    out_shape=jax.ShapeDtypeStruct((NP, D), jnp.float32),
)

_out_call = pl.pallas_call(
    _out_body,
    grid=(NP // ROWS_BLK,),
    in_specs=[_degp_spec(), _p_spec(), _rows_spec(), _bias_spec()],
    out_specs=_rows_spec(),
    out_shape=jax.ShapeDtypeStruct((N, D), jnp.float32),
)


def kernel(x, edge_index, W1, b1, W2, b2):
    pad2 = jnp.asarray(N + (np.arange(E_PAD - E, dtype=np.int32) % (NP - N))
                       ).reshape((E_PAD - E) // CW, CW)
    src2 = jnp.concatenate([edge_index[0].reshape(E // CW, CW), pad2])
    dst2 = jnp.concatenate([edge_index[1].reshape(E // CW, CW), pad2])
    zrows = jnp.zeros((RPT, D), jnp.float32)

    degp = _deg_call(edge_index[1])
    deg = degp.sum(axis=0).reshape(NP) + 1.0
    y1 = _lin1_call(deg, x, W1)
    p1 = _prop_call(y1, src2, dst2, zrows)
    y2 = _mid_call(deg, p1, y1, b1.reshape(1, D), W2)
    p2 = _prop_call(y2, src2, dst2, zrows)
    return _out_call(deg, p2, y2, b2.reshape(1, D))
